# baseline (speedup 1.0000x reference)
import jax
import jax.numpy as jnp
from jax import lax
from jax.experimental import pallas as pl
from jax.experimental.pallas import tpu as pltpu
from jax.experimental.pallas import tpu_sc as plsc

VOCAB = 1000000
CTX = 200
DIM = 32
BATCH = 4096

NC = 2
NS = 16
NW = NC * NS
ROWS_W = BATCH // NW
NB = 8
G = ROWS_W // NB
CH = NB * CTX


TBLK = 512
TGRID = -(-VOCAB // TBLK)


def _tc_transpose_body(tokT_ref, out_ref):
    x = tokT_ref[...]
    r = lax.broadcasted_iota(jnp.int32, (DIM, DIM), 0)
    c = lax.broadcasted_iota(jnp.int32, (DIM, DIM), 1)
    eye = (r == c).astype(jnp.float32)
    out_ref[:, pl.ds(0, DIM)] = lax.dot_general(
        x, eye, (((0,), (0,)), ((), ())),
        precision=lax.Precision.HIGHEST)


@jax.jit
def _tc_transpose(tokT):
    return pl.pallas_call(
        _tc_transpose_body,
        out_shape=jax.ShapeDtypeStruct((VOCAB, 128), jnp.float32),
        grid=(TGRID,),
        in_specs=[pl.BlockSpec((DIM, TBLK), lambda i: (0, i))],
        out_specs=pl.BlockSpec((TBLK, 128), lambda i: (i, 0)),
    )(tokT)


def _body(x_hbm, tok_hbm, pos_hbm, out_hbm, idx_v, rows_v, pos_v, sem):
    wid = lax.axis_index("s") * NC + lax.axis_index("c")
    base = wid * ROWS_W

    pltpu.sync_copy(pos_hbm, pos_v)

    for g in range(G):
        b0 = base + g * NB
        pltpu.sync_copy(x_hbm.at[pl.ds(b0, NB), :], idx_v)
        for sb in range(NB):
            pltpu.async_copy(tok_hbm.at[idx_v.at[sb]], rows_v.at[sb], sem)
        for sb in range(NB):
            pltpu.make_async_copy(tok_hbm.at[idx_v.at[sb]], rows_v.at[sb], sem).wait()

        def add_l(l, _):
            p0 = pos_v[l, pl.ds(0, 16)]
            p1 = pos_v[l, pl.ds(16, 16)]
            for sb in range(NB):
                rows_v[sb, l, pl.ds(0, 16)] = rows_v[sb, l, pl.ds(0, 16)] + p0
                rows_v[sb, l, pl.ds(16, 16)] = rows_v[sb, l, pl.ds(16, 16)] + p1
            return 0

        lax.fori_loop(0, CTX, add_l, 0, unroll=2)

        pltpu.sync_copy(rows_v, out_hbm.at[pl.ds(b0, NB), :, pl.ds(0, DIM)])


@jax.jit
def _embed(x, token_table, pos_table):
    mesh = plsc.VectorSubcoreMesh(core_axis_name="c", subcore_axis_name="s")
    padded = pl.kernel(
        _body,
        out_type=jax.ShapeDtypeStruct((BATCH, CTX, 128), jnp.float32),
        mesh=mesh,
        scratch_types=[
            pltpu.VMEM((NB, CTX), jnp.int32),
            pltpu.VMEM((NB, CTX, DIM), jnp.float32),
            pltpu.VMEM((CTX, DIM), jnp.float32),
            pltpu.SemaphoreType.DMA,
        ],
        compiler_params=pltpu.CompilerParams(use_tc_tiling_on_sc=False),
    )(x, token_table, pos_table)
    return lax.slice(padded, (0, 0, 0), (BATCH, CTX, DIM))


def kernel(x, token_table, pos_table):
    tok128 = _tc_transpose(token_table.T)
    tok4 = jnp.reshape(tok128, (4 * VOCAB, DIM))
    x4 = x.astype(jnp.int32) * 4
    return _embed(x4, tok4, pos_table)


# speedup vs baseline: 2.1814x; 2.1814x over previous
import jax
import jax.numpy as jnp
from jax import lax
from jax.experimental import pallas as pl
from jax.experimental.pallas import tpu as pltpu
from jax.experimental.pallas import tpu_sc as plsc

VOCAB = 1000000
CTX = 200
DIM = 32
BATCH = 4096

NC = 2
NS = 16
NW = NC * NS
ROWS_W = BATCH // NW
NB = 8
G = ROWS_W // NB
CH = NB * CTX


TBLK = 512
TGRID = -(-VOCAB // TBLK)


def _body(x_hbm, tok_hbm, pos_hbm, out_hbm, idx_v, rows_v, pos_v, sem):
    wid = lax.axis_index("s") * NC + lax.axis_index("c")
    base = wid * ROWS_W

    pltpu.sync_copy(pos_hbm, pos_v)

    for g in range(G):
        b0 = base + g * NB
        pltpu.sync_copy(x_hbm.at[pl.ds(b0, NB), :], idx_v)
        for sb in range(NB):
            pltpu.async_copy(tok_hbm.at[idx_v.at[sb]], rows_v.at[sb], sem)
        for sb in range(NB):
            pltpu.make_async_copy(tok_hbm.at[idx_v.at[sb]], rows_v.at[sb], sem).wait()

        def add_l(l, _):
            p0 = pos_v[l, pl.ds(0, 16)]
            p1 = pos_v[l, pl.ds(16, 16)]
            for sb in range(NB):
                rows_v[sb, l, pl.ds(0, 16)] = rows_v[sb, l, pl.ds(0, 16)] + p0
                rows_v[sb, l, pl.ds(16, 16)] = rows_v[sb, l, pl.ds(16, 16)] + p1
            return 0

        lax.fori_loop(0, CTX, add_l, 0, unroll=2)

        pltpu.sync_copy(rows_v, out_hbm.at[pl.ds(b0, NB), :, pl.ds(0, DIM)])


@jax.jit
def _embed(x, token_table, pos_table):
    mesh = plsc.VectorSubcoreMesh(core_axis_name="c", subcore_axis_name="s")
    padded = pl.kernel(
        _body,
        out_type=jax.ShapeDtypeStruct((BATCH, CTX, 128), jnp.float32),
        mesh=mesh,
        scratch_types=[
            pltpu.VMEM((NB, CTX), jnp.int32),
            pltpu.VMEM((NB, CTX, DIM), jnp.float32),
            pltpu.VMEM((CTX, DIM), jnp.float32),
            pltpu.SemaphoreType.DMA,
        ],
        compiler_params=pltpu.CompilerParams(use_tc_tiling_on_sc=False),
    )(x, token_table, pos_table)
    return lax.slice(padded, (0, 0, 0), (BATCH, CTX, DIM))


def kernel(x, token_table, pos_table):
    tok128 = jnp.pad(token_table, ((0, 0), (0, 128 - DIM)))
    tok4 = jnp.reshape(tok128, (4 * VOCAB, DIM))
    x4 = x.astype(jnp.int32) * 4
    return _embed(x4, tok4, pos_table)
